# baseline (speedup 1.0000x reference)
import functools

import jax
import jax.numpy as jnp
from jax import lax
from jax.experimental import pallas as pl
from jax.experimental.pallas import tpu as pltpu
from jax.experimental.pallas import tpu_sc as plsc

H = 200
W = 200
C = 256
R = 512
OUT_HW = 7
NBIN = OUT_HW * OUT_HW
NW = 32
RPW = R // NW
NSTEP = RPW * OUT_HW
NROW = 112
ROIBLK = C * NBIN

_DNUMS = lax.GatherDimensionNumbers(
    offset_dims=(), collapsed_slice_dims=(0,), start_index_map=(0,))


def _vtake(vec, idx):
    return lax.gather(vec, idx[:, None], _DNUMS, (1,),
                      mode=lax.GatherScatterMode.PROMISE_IN_BOUNDS)


def _sc_body(table, roisp, out, rois_v, idx0, idx1, rows0, rows1, ob,
             w0, w1, g0, g1, o0, o1):
    cid = lax.axis_index("c")
    sid = lax.axis_index("s")
    wid = sid * 2 + cid
    roi0 = wid * RPW
    pltpu.sync_copy(roisp.at[pl.ds(roi0, RPW)], rois_v)

    it = lax.iota(jnp.int32, 16)
    iy_pat = (it >> 3) & 1
    ix_pat = (it >> 2) & 1
    cy_m = ((it >> 1) & 1) == 1
    cx_m = (it & 1) == 1
    offs = (it >> 1).astype(jnp.float32) + ((it & 1).astype(jnp.float32) + 0.5) * 0.5
    chv = [32 * (v // 2) + 2 * it + (v % 2) for v in range(16)]

    def build(s, idx_buf, w_buf):
        i = s // OUT_HW
        p = s - i * OUT_HW
        vec = rois_v[i, :]
        b40k = _vtake(vec, jnp.zeros((16,), jnp.int32)).astype(jnp.int32) * (H * W)
        x1 = _vtake(vec, jnp.full((16,), 1, jnp.int32)) * 0.25 - 0.5
        y1 = _vtake(vec, jnp.full((16,), 2, jnp.int32)) * 0.25 - 0.5
        x2 = _vtake(vec, jnp.full((16,), 3, jnp.int32)) * 0.25 - 0.5
        y2 = _vtake(vec, jnp.full((16,), 4, jnp.int32)) * 0.25 - 0.5
        bh = (y2 - y1) * (1.0 / OUT_HW)
        bw = (x2 - x1) * (1.0 / OUT_HW)

        Yv = y1 + offs * bh
        vy = jnp.where((Yv >= -1.0) & (Yv <= 1.0 * H), 0.5, 0.0)
        ycl = jnp.clip(Yv, 0.0, H - 1.0)
        yq = ycl.astype(jnp.int32)
        ly = ycl - yq.astype(jnp.float32)
        ya = b40k + yq * W
        yb = b40k + jnp.minimum(yq + 1, H - 1) * W
        wyh = (1.0 - ly) * vy
        wyl = ly * vy

        Xv = x1 + offs * bw
        vx = jnp.where((Xv >= -1.0) & (Xv <= 1.0 * W), 0.5, 0.0)
        xcl = jnp.clip(Xv, 0.0, W - 1.0)
        xq = xcl.astype(jnp.int32)
        lx = xcl - xq.astype(jnp.float32)
        xb = jnp.minimum(xq + 1, W - 1)
        wxh = (1.0 - lx) * vx
        wxl = lx * vx

        lanes_t = 2 * p + iy_pat
        ysel = jnp.where(cy_m, _vtake(yb, lanes_t), _vtake(ya, lanes_t))
        wy_s = jnp.where(cy_m, _vtake(wyl, lanes_t), _vtake(wyh, lanes_t))
        for pw in range(OUT_HW):
            lanes_u = 2 * pw + ix_pat
            xsel = jnp.where(cx_m, _vtake(xb, lanes_u), _vtake(xq, lanes_u))
            wx_s = jnp.where(cx_m, _vtake(wxl, lanes_u), _vtake(wxh, lanes_u))
            idx_buf[pl.ds(pw * 16, 16)] = ysel + xsel
            w_buf[pl.ds(pw * 16, 16)] = wy_s * wx_s

    def consume(s, rows_buf, w_buf):
        i = s // OUT_HW
        p = s - i * OUT_HW
        par = i & 1
        chpar = [c + par * C for c in chv]

        @pl.when((p == 0) & (i >= 2) & (par == 0))
        def _():
            pltpu.make_async_copy(ob.at[pl.ds(0, C)], out.at[0], o0).wait()

        @pl.when((p == 0) & (i >= 2) & (par == 1))
        def _():
            pltpu.make_async_copy(ob.at[pl.ds(0, C)], out.at[0], o1).wait()

        def bin_body(pw, carry):
            w16 = w_buf[pl.ds(pw * 16, 16)]
            zero = jnp.zeros((16,), jnp.float32)

            def j_body(j, accs):
                wj = _vtake(w16, jnp.broadcast_to(j, (16,)))
                row = pw * 16 + j
                new = []
                for h in range(8):
                    ab = rows_buf[row, pl.ds(h * 32, 32)]
                    word = plsc.bitcast(ab, jnp.int32)
                    lo = plsc.bitcast(word << 16, jnp.float32)
                    hi = plsc.bitcast(word, jnp.float32)
                    new.append(accs[2 * h] + wj * lo)
                    new.append(accs[2 * h + 1] + wj * hi)
                return tuple(new)

            accs = lax.fori_loop(0, 16, j_body, (zero,) * 16)
            ps = jnp.broadcast_to(p, (16,))
            pws = jnp.broadcast_to(pw, (16,))
            for v in range(16):
                plsc.store_scatter(ob, [chpar[v], ps, pws], accs[v])
            return carry

        lax.fori_loop(0, OUT_HW, bin_body, 0)

        @pl.when((p == OUT_HW - 1) & (par == 0))
        def _():
            pltpu.async_copy(ob.at[pl.ds(0, C)], out.at[roi0 + i], o0)

        @pl.when((p == OUT_HW - 1) & (par == 1))
        def _():
            pltpu.async_copy(ob.at[pl.ds(C, C)], out.at[roi0 + i], o1)

    build(jnp.int32(0), idx0, w0)
    pltpu.async_copy(table.at[idx0], rows0, g0)

    def loop_body(g, carry):
        s = 2 * g
        build(s + 1, idx1, w1)
        pltpu.async_copy(table.at[idx1], rows1, g1)
        pltpu.make_async_copy(table.at[idx0], rows0, g0).wait()
        consume(s, rows0, w0)

        @pl.when(g < NSTEP // 2 - 1)
        def _():
            build(s + 2, idx0, w0)
            pltpu.async_copy(table.at[idx0], rows0, g0)

        pltpu.make_async_copy(table.at[idx1], rows1, g1).wait()
        consume(s + 1, rows1, w1)
        return carry

    lax.fori_loop(0, NSTEP // 2, loop_body, 0)
    pltpu.make_async_copy(ob.at[pl.ds(0, C)], out.at[0], o0).wait()
    pltpu.make_async_copy(ob.at[pl.ds(0, C)], out.at[0], o1).wait()


@functools.cache
def _sc_call():
    return functools.partial(
        pl.kernel,
        out_type=jax.ShapeDtypeStruct((R, C, OUT_HW, OUT_HW), jnp.float32),
        mesh=plsc.VectorSubcoreMesh(core_axis_name="c", subcore_axis_name="s"),
        compiler_params=pltpu.CompilerParams(use_tc_tiling_on_sc=False,
                                             needs_layout_passes=False),
        scratch_types=[
            pltpu.VMEM((RPW, 16), jnp.float32),
            pltpu.VMEM((NROW,), jnp.int32),
            pltpu.VMEM((NROW,), jnp.int32),
            pltpu.VMEM((NROW, C), jnp.bfloat16),
            pltpu.VMEM((NROW, C), jnp.bfloat16),
            pltpu.VMEM((2 * C, OUT_HW, OUT_HW), jnp.float32),
            pltpu.VMEM((NROW,), jnp.float32),
            pltpu.VMEM((NROW,), jnp.float32),
            pltpu.SemaphoreType.DMA,
            pltpu.SemaphoreType.DMA,
            pltpu.SemaphoreType.DMA,
            pltpu.SemaphoreType.DMA,
        ],
    )(_sc_body)


def kernel(input, rois):
    table = jnp.transpose(input, (0, 2, 3, 1)).reshape(2 * H * W, C)
    table = table.astype(jnp.bfloat16)
    roisp = jnp.pad(rois, ((0, 0), (0, 11)))
    return _sc_call()(table, roisp)


# speedup vs baseline: 1.4077x; 1.4077x over previous
import functools

import jax
import jax.numpy as jnp
from jax import lax
from jax.experimental import pallas as pl
from jax.experimental.pallas import tpu as pltpu
from jax.experimental.pallas import tpu_sc as plsc

H = 200
W = 200
C = 256
R = 512
OUT_HW = 7
NBIN = OUT_HW * OUT_HW
NW = 32
RPW = R // NW
NSTEP = RPW * OUT_HW
NROW = 112
ROIBLK = C * NBIN

_DNUMS = lax.GatherDimensionNumbers(
    offset_dims=(), collapsed_slice_dims=(0,), start_index_map=(0,))


def _vtake(vec, idx):
    return lax.gather(vec, idx[:, None], _DNUMS, (1,),
                      mode=lax.GatherScatterMode.PROMISE_IN_BOUNDS)


def _sc_body(table, roisp, out, rois_v, idx0, idx1, rows0, rows1, ob,
             w0, w1, g0, g1, o0, o1):
    cid = lax.axis_index("c")
    sid = lax.axis_index("s")
    wid = sid * 2 + cid
    roi0 = wid * RPW
    pltpu.sync_copy(roisp.at[pl.ds(roi0, RPW)], rois_v)

    it = lax.iota(jnp.int32, 16)
    iy_pat = (it >> 3) & 1
    ix_pat = (it >> 2) & 1
    cy_m = ((it >> 1) & 1) == 1
    cx_m = (it & 1) == 1
    offs = (it >> 1).astype(jnp.float32) + ((it & 1).astype(jnp.float32) + 0.5) * 0.5
    iota98 = it * (2 * NBIN)

    def build(s, idx_buf, w_buf):
        i = s // OUT_HW
        p = s - i * OUT_HW
        vec = rois_v[i, :]
        b40k = _vtake(vec, jnp.zeros((16,), jnp.int32)).astype(jnp.int32) * (H * W)
        x1 = _vtake(vec, jnp.full((16,), 1, jnp.int32)) * 0.25 - 0.5
        y1 = _vtake(vec, jnp.full((16,), 2, jnp.int32)) * 0.25 - 0.5
        x2 = _vtake(vec, jnp.full((16,), 3, jnp.int32)) * 0.25 - 0.5
        y2 = _vtake(vec, jnp.full((16,), 4, jnp.int32)) * 0.25 - 0.5
        bh = (y2 - y1) * (1.0 / OUT_HW)
        bw = (x2 - x1) * (1.0 / OUT_HW)

        Yv = y1 + offs * bh
        vy = jnp.where((Yv >= -1.0) & (Yv <= 1.0 * H), 0.5, 0.0)
        ycl = jnp.clip(Yv, 0.0, H - 1.0)
        yq = ycl.astype(jnp.int32)
        ly = ycl - yq.astype(jnp.float32)
        ya = b40k + yq * W
        yb = b40k + jnp.minimum(yq + 1, H - 1) * W
        wyh = (1.0 - ly) * vy
        wyl = ly * vy

        Xv = x1 + offs * bw
        vx = jnp.where((Xv >= -1.0) & (Xv <= 1.0 * W), 0.5, 0.0)
        xcl = jnp.clip(Xv, 0.0, W - 1.0)
        xq = xcl.astype(jnp.int32)
        lx = xcl - xq.astype(jnp.float32)
        xb = jnp.minimum(xq + 1, W - 1)
        wxh = (1.0 - lx) * vx
        wxl = lx * vx

        lanes_t = 2 * p + iy_pat
        ysel = jnp.where(cy_m, _vtake(yb, lanes_t), _vtake(ya, lanes_t))
        wy_s = jnp.where(cy_m, _vtake(wyl, lanes_t), _vtake(wyh, lanes_t))
        for pw in range(OUT_HW):
            lanes_u = 2 * pw + ix_pat
            xsel = jnp.where(cx_m, _vtake(xb, lanes_u), _vtake(xq, lanes_u))
            wx_s = jnp.where(cx_m, _vtake(wxl, lanes_u), _vtake(wxh, lanes_u))
            idx_buf[pl.ds(pw * 16, 16)] = ysel + xsel
            w_buf[pl.ds(pw * 16, 16)] = wy_s * wx_s

    def consume(s, rows_buf, w_buf):
        i = s // OUT_HW
        p = s - i * OUT_HW
        par = i & 1
        pbase = par * ROIBLK

        @pl.when((p == 0) & (i >= 2) & (par == 0))
        def _():
            pltpu.make_async_copy(ob.at[pl.ds(0, ROIBLK)], out.at[0], o0).wait()

        @pl.when((p == 0) & (i >= 2) & (par == 1))
        def _():
            pltpu.make_async_copy(ob.at[pl.ds(0, ROIBLK)], out.at[0], o1).wait()

        def bin_body(pw, carry):
            w16 = w_buf[pl.ds(pw * 16, 16)]
            zero = jnp.zeros((16,), jnp.float32)

            def j_body(j, accs):
                wj = _vtake(w16, jnp.broadcast_to(j, (16,)))
                row = pw * 16 + j
                new = []
                for h in range(8):
                    ab = rows_buf[row, pl.ds(h * 32, 32)]
                    word = plsc.bitcast(ab, jnp.int32)
                    lo = plsc.bitcast(word << 16, jnp.float32)
                    hi = plsc.bitcast(word, jnp.float32)
                    new.append(accs[2 * h] + wj * lo)
                    new.append(accs[2 * h + 1] + wj * hi)
                return tuple(new)

            accs = lax.fori_loop(0, 16, j_body, (zero,) * 16)
            basev = iota98 + (pbase + p * OUT_HW + pw)
            for v in range(16):
                off = NBIN * 32 * (v // 2) + NBIN * (v % 2)
                plsc.store_scatter(ob, [basev + off], accs[v])
            return carry

        lax.fori_loop(0, OUT_HW, bin_body, 0)

        @pl.when((p == OUT_HW - 1) & (par == 0))
        def _():
            pltpu.async_copy(ob.at[pl.ds(0, ROIBLK)], out.at[roi0 + i], o0)

        @pl.when((p == OUT_HW - 1) & (par == 1))
        def _():
            pltpu.async_copy(ob.at[pl.ds(ROIBLK, ROIBLK)], out.at[roi0 + i], o1)

    build(jnp.int32(0), idx0, w0)
    pltpu.async_copy(table.at[idx0], rows0, g0)

    def loop_body(g, carry):
        s = 2 * g
        build(s + 1, idx1, w1)
        pltpu.async_copy(table.at[idx1], rows1, g1)
        pltpu.make_async_copy(table.at[idx0], rows0, g0).wait()
        consume(s, rows0, w0)

        @pl.when(g < NSTEP // 2 - 1)
        def _():
            build(s + 2, idx0, w0)
            pltpu.async_copy(table.at[idx0], rows0, g0)

        pltpu.make_async_copy(table.at[idx1], rows1, g1).wait()
        consume(s + 1, rows1, w1)
        return carry

    lax.fori_loop(0, NSTEP // 2, loop_body, 0)
    pltpu.make_async_copy(ob.at[pl.ds(0, ROIBLK)], out.at[0], o0).wait()
    pltpu.make_async_copy(ob.at[pl.ds(0, ROIBLK)], out.at[0], o1).wait()


def _tc_transpose_body(x_ref, o_ref):
    for yy in range(8):
        blk = x_ref[0, :, yy, :]
        o_ref[pl.ds(yy * W, W), :] = jnp.transpose(blk, (1, 0)).astype(jnp.bfloat16)


@functools.cache
def _tc_transpose():
    return pl.pallas_call(
        _tc_transpose_body,
        grid=(2, H // 8),
        in_specs=[pl.BlockSpec((1, C, 8, W), lambda b, y: (b, 0, y, 0))],
        out_specs=pl.BlockSpec((8 * W, C), lambda b, y: (b * (H // 8) + y, 0)),
        out_shape=jax.ShapeDtypeStruct((2 * H * W, C), jnp.bfloat16),
    )


@functools.cache
def _sc_call():
    return functools.partial(
        pl.kernel,
        out_type=jax.ShapeDtypeStruct((R, ROIBLK), jnp.float32),
        mesh=plsc.VectorSubcoreMesh(core_axis_name="c", subcore_axis_name="s"),
        compiler_params=pltpu.CompilerParams(use_tc_tiling_on_sc=False,
                                             needs_layout_passes=False),
        scratch_types=[
            pltpu.VMEM((RPW, 16), jnp.float32),
            pltpu.VMEM((NROW,), jnp.int32),
            pltpu.VMEM((NROW,), jnp.int32),
            pltpu.VMEM((NROW, C), jnp.bfloat16),
            pltpu.VMEM((NROW, C), jnp.bfloat16),
            pltpu.VMEM((2 * ROIBLK,), jnp.float32),
            pltpu.VMEM((NROW,), jnp.float32),
            pltpu.VMEM((NROW,), jnp.float32),
            pltpu.SemaphoreType.DMA,
            pltpu.SemaphoreType.DMA,
            pltpu.SemaphoreType.DMA,
            pltpu.SemaphoreType.DMA,
        ],
    )(_sc_body)


def kernel(input, rois):
    table = _tc_transpose()(input)
    roisp = jnp.pad(rois, ((0, 0), (0, 11)))
    out = _sc_call()(table, roisp)
    return out.reshape(R, C, OUT_HW, OUT_HW)


# speedup vs baseline: 1.8923x; 1.3443x over previous
import functools

import jax
import jax.numpy as jnp
from jax import lax
from jax.experimental import pallas as pl
from jax.experimental.pallas import tpu as pltpu
from jax.experimental.pallas import tpu_sc as plsc

H = 200
W = 200
C = 256
R = 512
OUT_HW = 7
NBIN = OUT_HW * OUT_HW
NW = 32
RPW = R // NW
NSTEP = RPW * OUT_HW
NROW = 112
ROIBLK = C * NBIN

_DNUMS = lax.GatherDimensionNumbers(
    offset_dims=(), collapsed_slice_dims=(0,), start_index_map=(0,))


def _vtake(vec, idx):
    return lax.gather(vec, idx[:, None], _DNUMS, (1,),
                      mode=lax.GatherScatterMode.PROMISE_IN_BOUNDS)


def _sc_body(table, roisp, out, rois_v, idx0, idx1, rows0, rows1, ob,
             w0, w1, g0, g1, o0, o1):
    cid = lax.axis_index("c")
    sid = lax.axis_index("s")
    wid = sid * 2 + cid
    roi0 = wid * RPW
    pltpu.sync_copy(roisp.at[pl.ds(roi0, RPW)], rois_v.at[:, pl.ds(0, 5)])

    it = lax.iota(jnp.int32, 16)
    iy_pat = (it >> 3) & 1
    ix_pat = (it >> 2) & 1
    cy_m = ((it >> 1) & 1) == 1
    cx_m = (it & 1) == 1
    offs = (it >> 1).astype(jnp.float32) + ((it & 1).astype(jnp.float32) + 0.5) * 0.5
    iota98 = it * (2 * NBIN)

    def build(s, idx_buf, w_buf):
        i = s // OUT_HW
        p = s - i * OUT_HW
        vec = rois_v[i, :]
        b40k = _vtake(vec, jnp.zeros((16,), jnp.int32)).astype(jnp.int32) * (H * W)
        x1 = _vtake(vec, jnp.full((16,), 1, jnp.int32)) * 0.25 - 0.5
        y1 = _vtake(vec, jnp.full((16,), 2, jnp.int32)) * 0.25 - 0.5
        x2 = _vtake(vec, jnp.full((16,), 3, jnp.int32)) * 0.25 - 0.5
        y2 = _vtake(vec, jnp.full((16,), 4, jnp.int32)) * 0.25 - 0.5
        bh = (y2 - y1) * (1.0 / OUT_HW)
        bw = (x2 - x1) * (1.0 / OUT_HW)

        Yv = y1 + offs * bh
        vy = jnp.where((Yv >= -1.0) & (Yv <= 1.0 * H), 0.5, 0.0)
        ycl = jnp.clip(Yv, 0.0, H - 1.0)
        yq = ycl.astype(jnp.int32)
        ly = ycl - yq.astype(jnp.float32)
        ya = b40k + yq * W
        yb = b40k + jnp.minimum(yq + 1, H - 1) * W
        wyh = (1.0 - ly) * vy
        wyl = ly * vy

        Xv = x1 + offs * bw
        vx = jnp.where((Xv >= -1.0) & (Xv <= 1.0 * W), 0.5, 0.0)
        xcl = jnp.clip(Xv, 0.0, W - 1.0)
        xq = xcl.astype(jnp.int32)
        lx = xcl - xq.astype(jnp.float32)
        xb = jnp.minimum(xq + 1, W - 1)
        wxh = (1.0 - lx) * vx
        wxl = lx * vx

        lanes_t = 2 * p + iy_pat
        ysel = jnp.where(cy_m, _vtake(yb, lanes_t), _vtake(ya, lanes_t))
        wy_s = jnp.where(cy_m, _vtake(wyl, lanes_t), _vtake(wyh, lanes_t))
        for pw in range(OUT_HW):
            lanes_u = 2 * pw + ix_pat
            xsel = jnp.where(cx_m, _vtake(xb, lanes_u), _vtake(xq, lanes_u))
            wx_s = jnp.where(cx_m, _vtake(wxl, lanes_u), _vtake(wxh, lanes_u))
            idx_buf[pl.ds(pw * 16, 16)] = ysel + xsel
            w_buf[pl.ds(pw * 16, 16)] = wy_s * wx_s

    def consume(s, rows_buf, w_buf):
        i = s // OUT_HW
        p = s - i * OUT_HW
        par = i & 1
        pbase = par * ROIBLK

        @pl.when((p == 0) & (i >= 2) & (par == 0))
        def _():
            pltpu.make_async_copy(ob.at[pl.ds(0, ROIBLK)], out.at[0], o0).wait()

        @pl.when((p == 0) & (i >= 2) & (par == 1))
        def _():
            pltpu.make_async_copy(ob.at[pl.ds(0, ROIBLK)], out.at[0], o1).wait()

        def bin_body(pw, carry):
            w16 = w_buf[pl.ds(pw * 16, 16)]
            zero = jnp.zeros((16,), jnp.float32)

            def j_body(j, accs):
                new = list(accs)
                for u in range(2):
                    wj = _vtake(w16, jnp.broadcast_to(2 * j + u, (16,)))
                    row = pw * 16 + 2 * j + u
                    for h in range(8):
                        ab = rows_buf[row, pl.ds(h * 32, 32)]
                        word = plsc.bitcast(ab, jnp.int32)
                        lo = plsc.bitcast(word << 16, jnp.float32)
                        hi = plsc.bitcast(word, jnp.float32)
                        new[2 * h] = new[2 * h] + wj * lo
                        new[2 * h + 1] = new[2 * h + 1] + wj * hi
                return tuple(new)

            accs = lax.fori_loop(0, 8, j_body, (zero,) * 16)
            basev = iota98 + (pbase + p * OUT_HW + pw)
            for v in range(16):
                off = NBIN * 32 * (v // 2) + NBIN * (v % 2)
                plsc.store_scatter(ob, [basev + off], accs[v])
            return carry

        lax.fori_loop(0, OUT_HW, bin_body, 0)

        @pl.when((p == OUT_HW - 1) & (par == 0))
        def _():
            pltpu.async_copy(ob.at[pl.ds(0, ROIBLK)], out.at[roi0 + i], o0)

        @pl.when((p == OUT_HW - 1) & (par == 1))
        def _():
            pltpu.async_copy(ob.at[pl.ds(ROIBLK, ROIBLK)], out.at[roi0 + i], o1)

    build(jnp.int32(0), idx0, w0)
    pltpu.async_copy(table.at[idx0], rows0, g0)

    def loop_body(g, carry):
        s = 2 * g
        build(s + 1, idx1, w1)
        pltpu.async_copy(table.at[idx1], rows1, g1)
        pltpu.make_async_copy(table.at[idx0], rows0, g0).wait()
        consume(s, rows0, w0)

        @pl.when(g < NSTEP // 2 - 1)
        def _():
            build(s + 2, idx0, w0)
            pltpu.async_copy(table.at[idx0], rows0, g0)

        pltpu.make_async_copy(table.at[idx1], rows1, g1).wait()
        consume(s + 1, rows1, w1)
        return carry

    lax.fori_loop(0, NSTEP // 2, loop_body, 0)
    pltpu.make_async_copy(ob.at[pl.ds(0, ROIBLK)], out.at[0], o0).wait()
    pltpu.make_async_copy(ob.at[pl.ds(0, ROIBLK)], out.at[0], o1).wait()


@functools.cache
def _sc_call():
    return functools.partial(
        pl.kernel,
        out_type=jax.ShapeDtypeStruct((R, ROIBLK), jnp.float32),
        mesh=plsc.VectorSubcoreMesh(core_axis_name="c", subcore_axis_name="s"),
        compiler_params=pltpu.CompilerParams(use_tc_tiling_on_sc=False,
                                             needs_layout_passes=False),
        scratch_types=[
            pltpu.VMEM((RPW, 16), jnp.float32),
            pltpu.VMEM((NROW,), jnp.int32),
            pltpu.VMEM((NROW,), jnp.int32),
            pltpu.VMEM((NROW, C), jnp.bfloat16),
            pltpu.VMEM((NROW, C), jnp.bfloat16),
            pltpu.VMEM((2 * ROIBLK,), jnp.float32),
            pltpu.VMEM((NROW,), jnp.float32),
            pltpu.VMEM((NROW,), jnp.float32),
            pltpu.SemaphoreType.DMA,
            pltpu.SemaphoreType.DMA,
            pltpu.SemaphoreType.DMA,
            pltpu.SemaphoreType.DMA,
        ],
    )(_sc_body)


def kernel(input, rois):
    table = jnp.transpose(input, (0, 2, 3, 1)).reshape(2 * H * W, C)
    table = table.astype(jnp.bfloat16)
    out = _sc_call()(table, rois)
    return out.reshape(R, C, OUT_HW, OUT_HW)


# speedup vs baseline: 1.9469x; 1.0289x over previous
import functools

import jax
import jax.numpy as jnp
from jax import lax
from jax.experimental import pallas as pl
from jax.experimental.pallas import tpu as pltpu
from jax.experimental.pallas import tpu_sc as plsc

H = 200
W = 200
C = 256
R = 512
OUT_HW = 7
NBIN = OUT_HW * OUT_HW
NW = 32
RPW = R // NW
NSTEP = RPW * OUT_HW
NROW = 112
ROIBLK = C * NBIN

_DNUMS = lax.GatherDimensionNumbers(
    offset_dims=(), collapsed_slice_dims=(0,), start_index_map=(0,))


def _vtake(vec, idx):
    return lax.gather(vec, idx[:, None], _DNUMS, (1,),
                      mode=lax.GatherScatterMode.PROMISE_IN_BOUNDS)


def _sc_body(table, roisp, out, rois_v, idx0a, idx0b, idx1a, idx1b,
             rows0a, rows0b, rows1a, rows1b, ob, w0, w1,
             g0a, g0b, g1a, g1b, o0, o1):
    cid = lax.axis_index("c")
    sid = lax.axis_index("s")
    wid = sid * 2 + cid
    roi0 = wid * RPW
    pltpu.sync_copy(roisp.at[pl.ds(roi0, RPW)], rois_v)

    it = lax.iota(jnp.int32, 16)
    iy_pat = (it >> 3) & 1
    ix_pat = (it >> 2) & 1
    cy_m = ((it >> 1) & 1) == 1
    cx_m = (it & 1) == 1
    offs = (it >> 1).astype(jnp.float32) + ((it & 1).astype(jnp.float32) + 0.5) * 0.5
    iota98 = it * (2 * NBIN)

    def build(s, idx_buf, w_buf):
        i = s // OUT_HW
        p = s - i * OUT_HW
        vec = rois_v[i, :]
        b40k = _vtake(vec, jnp.zeros((16,), jnp.int32)).astype(jnp.int32) * (H * W)
        x1 = _vtake(vec, jnp.full((16,), 1, jnp.int32)) * 0.25 - 0.5
        y1 = _vtake(vec, jnp.full((16,), 2, jnp.int32)) * 0.25 - 0.5
        x2 = _vtake(vec, jnp.full((16,), 3, jnp.int32)) * 0.25 - 0.5
        y2 = _vtake(vec, jnp.full((16,), 4, jnp.int32)) * 0.25 - 0.5
        bh = (y2 - y1) * (1.0 / OUT_HW)
        bw = (x2 - x1) * (1.0 / OUT_HW)

        Yv = y1 + offs * bh
        vy = jnp.where((Yv >= -1.0) & (Yv <= 1.0 * H), 0.5, 0.0)
        ycl = jnp.clip(Yv, 0.0, H - 1.0)
        yq = ycl.astype(jnp.int32)
        ly = ycl - yq.astype(jnp.float32)
        ya = b40k + yq * W
        yb = b40k + jnp.minimum(yq + 1, H - 1) * W
        wyh = (1.0 - ly) * vy
        wyl = ly * vy

        Xv = x1 + offs * bw
        vx = jnp.where((Xv >= -1.0) & (Xv <= 1.0 * W), 0.5, 0.0)
        xcl = jnp.clip(Xv, 0.0, W - 1.0)
        xq = xcl.astype(jnp.int32)
        lx = xcl - xq.astype(jnp.float32)
        xb = jnp.minimum(xq + 1, W - 1)
        wxh = (1.0 - lx) * vx
        wxl = lx * vx

        lanes_t = 2 * p + iy_pat
        ysel = jnp.where(cy_m, _vtake(yb, lanes_t), _vtake(ya, lanes_t))
        wy_s = jnp.where(cy_m, _vtake(wyl, lanes_t), _vtake(wyh, lanes_t))
        idx_a, idx_b = idx_buf
        for pw in range(OUT_HW):
            lanes_u = 2 * pw + ix_pat
            xsel = jnp.where(cx_m, _vtake(xb, lanes_u), _vtake(xq, lanes_u))
            wx_s = jnp.where(cx_m, _vtake(wxl, lanes_u), _vtake(wxh, lanes_u))
            if pw < 4:
                idx_a[pl.ds(pw * 16, 16)] = ysel + xsel
            else:
                idx_b[pl.ds((pw - 4) * 16, 16)] = ysel + xsel
            w_buf[pl.ds(pw * 16, 16)] = wy_s * wx_s

    def consume(s, rows_a, rows_b, w_buf):
        i = s // OUT_HW
        p = s - i * OUT_HW
        par = i & 1
        pbase = par * ROIBLK

        @pl.when((p == 0) & (i >= 2) & (par == 0))
        def _():
            pltpu.make_async_copy(ob.at[pl.ds(0, ROIBLK)], out.at[0], o0).wait()

        @pl.when((p == 0) & (i >= 2) & (par == 1))
        def _():
            pltpu.make_async_copy(ob.at[pl.ds(0, ROIBLK)], out.at[0], o1).wait()

        def make_bin_body(rows_buf, pw_off):
          def bin_body(pw, carry):
            w16 = w_buf[pl.ds(pw * 16, 16)]
            zero = jnp.zeros((16,), jnp.float32)

            def j_body(j, accs):
                wj = _vtake(w16, jnp.broadcast_to(j, (16,)))
                row = (pw - pw_off) * 16 + j
                new = []
                for h in range(8):
                    ab = rows_buf[row, pl.ds(h * 32, 32)]
                    word = plsc.bitcast(ab, jnp.int32)
                    lo = plsc.bitcast(word << 16, jnp.float32)
                    hi = plsc.bitcast(word, jnp.float32)
                    new.append(accs[2 * h] + wj * lo)
                    new.append(accs[2 * h + 1] + wj * hi)
                return tuple(new)

            accs = lax.fori_loop(0, 16, j_body, (zero,) * 16)
            basev = iota98 + (pbase + p * OUT_HW + pw)
            for v in range(16):
                off = NBIN * 32 * (v // 2) + NBIN * (v % 2)
                plsc.store_scatter(ob, [basev + off], accs[v])
            return carry
          return bin_body

        lax.fori_loop(0, 4, make_bin_body(rows_a, 0), 0)
        lax.fori_loop(4, OUT_HW, make_bin_body(rows_b, 4), 0)

        @pl.when((p == OUT_HW - 1) & (par == 0))
        def _():
            pltpu.async_copy(ob.at[pl.ds(0, ROIBLK)], out.at[roi0 + i], o0)

        @pl.when((p == OUT_HW - 1) & (par == 1))
        def _():
            pltpu.async_copy(ob.at[pl.ds(ROIBLK, ROIBLK)], out.at[roi0 + i], o1)

    build(jnp.int32(0), (idx0a, idx0b), w0)
    pltpu.async_copy(table.at[idx0a], rows0a, g0a)
    pltpu.async_copy(table.at[idx0b], rows0b, g0b)

    def loop_body(g, carry):
        s = 2 * g
        build(s + 1, (idx1a, idx1b), w1)
        pltpu.async_copy(table.at[idx1a], rows1a, g1a)
        pltpu.async_copy(table.at[idx1b], rows1b, g1b)
        pltpu.make_async_copy(table.at[idx0a], rows0a, g0a).wait()
        pltpu.make_async_copy(table.at[idx0b], rows0b, g0b).wait()
        consume(s, rows0a, rows0b, w0)

        @pl.when(g < NSTEP // 2 - 1)
        def _():
            build(s + 2, (idx0a, idx0b), w0)
            pltpu.async_copy(table.at[idx0a], rows0a, g0a)
            pltpu.async_copy(table.at[idx0b], rows0b, g0b)

        pltpu.make_async_copy(table.at[idx1a], rows1a, g1a).wait()
        pltpu.make_async_copy(table.at[idx1b], rows1b, g1b).wait()
        consume(s + 1, rows1a, rows1b, w1)
        return carry

    lax.fori_loop(0, NSTEP // 2, loop_body, 0)
    pltpu.make_async_copy(ob.at[pl.ds(0, ROIBLK)], out.at[0], o0).wait()
    pltpu.make_async_copy(ob.at[pl.ds(0, ROIBLK)], out.at[0], o1).wait()


@functools.cache
def _sc_call():
    return functools.partial(
        pl.kernel,
        out_type=jax.ShapeDtypeStruct((R, ROIBLK), jnp.float32),
        mesh=plsc.VectorSubcoreMesh(core_axis_name="c", subcore_axis_name="s"),
        compiler_params=pltpu.CompilerParams(use_tc_tiling_on_sc=False,
                                             needs_layout_passes=False),
        scratch_types=[
            pltpu.VMEM((RPW, 16), jnp.float32),
            pltpu.VMEM((64,), jnp.int32),
            pltpu.VMEM((48,), jnp.int32),
            pltpu.VMEM((64,), jnp.int32),
            pltpu.VMEM((48,), jnp.int32),
            pltpu.VMEM((64, C), jnp.bfloat16),
            pltpu.VMEM((48, C), jnp.bfloat16),
            pltpu.VMEM((64, C), jnp.bfloat16),
            pltpu.VMEM((48, C), jnp.bfloat16),
            pltpu.VMEM((2 * ROIBLK,), jnp.float32),
            pltpu.VMEM((NROW,), jnp.float32),
            pltpu.VMEM((NROW,), jnp.float32),
            pltpu.SemaphoreType.DMA,
            pltpu.SemaphoreType.DMA,
            pltpu.SemaphoreType.DMA,
            pltpu.SemaphoreType.DMA,
            pltpu.SemaphoreType.DMA,
            pltpu.SemaphoreType.DMA,
        ],
    )(_sc_body)


def kernel(input, rois):
    table = jnp.transpose(input, (0, 2, 3, 1)).reshape(2 * H * W, C)
    table = table.astype(jnp.bfloat16)
    roisp = jnp.pad(rois, ((0, 0), (0, 11)))
    out = _sc_call()(table, roisp)
    return out.reshape(R, C, OUT_HW, OUT_HW)
